# speedup vs baseline: 14.1868x; 14.1868x over previous
import functools

import jax
import jax.numpy as jnp
from jax import lax
from jax.experimental import pallas as pl
from jax.experimental.pallas import tpu as pltpu
from jax.experimental.pallas import tpu_sc as plsc

B, T, E = 8, 2048, 256
S, WMAX = 512, 10
H = 150
CHUNK = 256

_NC, _NS = 2, 16
_NW = _NC * _NS


def _attn_prefix_body(x_ref, w1_ref, b1_ref, w2_ref, b2_ref, w3_ref, b3_ref,
                      q_ref):
    x = x_ref[0]
    h = jnp.maximum(
        jnp.dot(x, w1_ref[...], preferred_element_type=jnp.float32)
        + b1_ref[...], 0.0)
    h = jnp.maximum(
        jnp.dot(h, w2_ref[...], preferred_element_type=jnp.float32)
        + b2_ref[...], 0.0)
    a = (jnp.dot(h, w3_ref[...], preferred_element_type=jnp.float32)
         + b3_ref[...])
    z = x * a

    tri = (lax.broadcasted_iota(jnp.int32, (CHUNK, CHUNK), 0)
           > lax.broadcasted_iota(jnp.int32, (CHUNK, CHUNK), 1)
           ).astype(jnp.float32)
    carry = jnp.zeros((1, E), jnp.float32)
    for i in range(T // CHUNK):
        zc = z[i * CHUNK:(i + 1) * CHUNK, :]
        q_ref[0, i * CHUNK:(i + 1) * CHUNK, :] = (
            jnp.dot(tri, zc, preferred_element_type=jnp.float32) + carry)
        carry = carry + jnp.sum(zc, axis=0, keepdims=True)


def _attn_prefix(batch_embeds, w1, b1, w2, b2, w3, b3):
    return pl.pallas_call(
        _attn_prefix_body,
        grid=(B,),
        in_specs=[
            pl.BlockSpec((1, T, E), lambda b: (b, 0, 0)),
            pl.BlockSpec((E, H), lambda b: (0, 0)),
            pl.BlockSpec((1, H), lambda b: (0, 0)),
            pl.BlockSpec((H, H), lambda b: (0, 0)),
            pl.BlockSpec((1, H), lambda b: (0, 0)),
            pl.BlockSpec((H, 1), lambda b: (0, 0)),
            pl.BlockSpec((1, 1), lambda b: (0, 0)),
        ],
        out_specs=pl.BlockSpec((1, T, E), lambda b: (b, 0, 0)),
        out_shape=jax.ShapeDtypeStruct((B, T, E), jnp.float32),
    )(batch_embeds, w1, b1.reshape(1, H), w2, b2.reshape(1, H), w3,
      b3.reshape(1, 1))


def _sc_gather(table, idx):
    n, d = idx.shape[0], table.shape[1]
    per_w = n // _NW
    mesh = plsc.VectorSubcoreMesh(core_axis_name="c", subcore_axis_name="s")

    @functools.partial(
        pl.kernel,
        mesh=mesh,
        out_type=jax.ShapeDtypeStruct((n, d), table.dtype),
        scratch_types=[
            pltpu.VMEM((per_w,), jnp.int32),
            pltpu.VMEM((per_w, d), table.dtype),
            pltpu.SemaphoreType.DMA,
        ],
    )
    def k(table_hbm, idx_hbm, out_hbm, idx_v, rows_v, sem):
        wid = lax.axis_index("s") * _NC + lax.axis_index("c")
        base = wid * per_w
        pltpu.sync_copy(idx_hbm.at[pl.ds(base, per_w)], idx_v)
        pltpu.async_copy(table_hbm.at[idx_v], rows_v, sem).wait()
        pltpu.sync_copy(rows_v, out_hbm.at[pl.ds(base, per_w)])

    return k(table, idx)


_ROWS = 512


def _mention_body(gs_ref, ge_ref, qs_ref, qe_ref, w1_ref, b1_ref, w2_ref,
                  b2_ref, w3_ref, b3_ref, se_ref, ms_ref):
    gs = gs_ref[...]
    ge = ge_ref[...]
    w = qe_ref[...] - qs_ref[...]
    se = jnp.concatenate([gs, ge, w], axis=1)
    se_ref[...] = se
    h = jnp.maximum(
        jnp.dot(se, w1_ref[...], preferred_element_type=jnp.float32)
        + b1_ref[...], 0.0)
    h = jnp.maximum(
        jnp.dot(h, w2_ref[...], preferred_element_type=jnp.float32)
        + b2_ref[...], 0.0)
    ms_ref[...] = (jnp.dot(h, w3_ref[...], preferred_element_type=jnp.float32)
                   + b3_ref[...])


def _mention(gs, ge, qs, qe, w1, b1, w2, b2, w3, b3):
    n = gs.shape[0]
    row_spec = pl.BlockSpec((_ROWS, E), lambda i: (i, 0))
    return pl.pallas_call(
        _mention_body,
        grid=(n // _ROWS,),
        in_specs=[
            row_spec, row_spec, row_spec, row_spec,
            pl.BlockSpec((3 * E, H), lambda i: (0, 0)),
            pl.BlockSpec((1, H), lambda i: (0, 0)),
            pl.BlockSpec((H, H), lambda i: (0, 0)),
            pl.BlockSpec((1, H), lambda i: (0, 0)),
            pl.BlockSpec((H, 1), lambda i: (0, 0)),
            pl.BlockSpec((1, 1), lambda i: (0, 0)),
        ],
        out_specs=[
            pl.BlockSpec((_ROWS, 3 * E), lambda i: (i, 0)),
            pl.BlockSpec((_ROWS, 1), lambda i: (i, 0)),
        ],
        out_shape=[
            jax.ShapeDtypeStruct((n, 3 * E), jnp.float32),
            jax.ShapeDtypeStruct((n, 1), jnp.float32),
        ],
    )(gs, ge, qs, qe, w1, b1.reshape(1, H), w2, b2.reshape(1, H), w3,
      b3.reshape(1, 1))


def kernel(batch_embeds, span_starts, span_widths, Wa1, ba1, Wa2, ba2, Wa3,
           ba3, Ws1, bs1, Ws2, bs2, Ws3, bs3):
    starts = span_starts.astype(jnp.int32)
    ends = starts + span_widths.astype(jnp.int32)
    boff = (jnp.arange(B, dtype=jnp.int32) * T)[:, None]
    fs = (starts + boff).reshape(-1)
    fe = (ends + boff).reshape(-1)
    idx_emb = jnp.concatenate([fs, fe])
    idx_q = jnp.concatenate([fs, fe + 1])

    emb_flat = batch_embeds.reshape(B * T, E)
    g_emb = _sc_gather(emb_flat, idx_emb)

    q = _attn_prefix(batch_embeds, Wa1, ba1, Wa2, ba2, Wa3, ba3)
    g_q = _sc_gather(q.reshape(B * T, E), idx_q)

    n = B * S
    span_embeds, scores = _mention(g_emb[:n], g_emb[n:], g_q[:n], g_q[n:],
                                   Ws1, bs1, Ws2, bs2, Ws3, bs3)
    return span_embeds.reshape(B, S, 3 * E), scores.reshape(B, S, 1)


# speedup vs baseline: 14.8321x; 1.0455x over previous
import functools

import jax
import jax.numpy as jnp
from jax import lax
from jax.experimental import pallas as pl
from jax.experimental.pallas import tpu as pltpu
from jax.experimental.pallas import tpu_sc as plsc

B, T, E = 8, 2048, 256
S, WMAX = 512, 10
H = 150
CHUNK = 256

_NC, _NS = 2, 16
_NW = _NC * _NS


def _attn_prefix_body(x_ref, w1_ref, b1_ref, w2_ref, b2_ref, w3_ref, b3_ref,
                      q_ref):
    x = x_ref[0]
    h = jnp.maximum(
        jnp.dot(x, w1_ref[...], preferred_element_type=jnp.float32)
        + b1_ref[...], 0.0)
    h = jnp.maximum(
        jnp.dot(h, w2_ref[...], preferred_element_type=jnp.float32)
        + b2_ref[...], 0.0)
    a = (jnp.dot(h, w3_ref[...], preferred_element_type=jnp.float32)
         + b3_ref[...])
    z = x * a

    tri = (lax.broadcasted_iota(jnp.int32, (CHUNK, CHUNK), 0)
           > lax.broadcasted_iota(jnp.int32, (CHUNK, CHUNK), 1)
           ).astype(jnp.float32)
    carry = jnp.zeros((1, E), jnp.float32)
    for i in range(T // CHUNK):
        zc = z[i * CHUNK:(i + 1) * CHUNK, :]
        q_ref[0, i * CHUNK:(i + 1) * CHUNK, :] = (
            jnp.dot(tri, zc, preferred_element_type=jnp.float32) + carry)
        carry = carry + jnp.sum(zc, axis=0, keepdims=True)


def _attn_prefix(batch_embeds, w1, b1, w2, b2, w3, b3):
    return pl.pallas_call(
        _attn_prefix_body,
        grid=(B,),
        in_specs=[
            pl.BlockSpec((1, T, E), lambda b: (b, 0, 0)),
            pl.BlockSpec((E, H), lambda b: (0, 0)),
            pl.BlockSpec((1, H), lambda b: (0, 0)),
            pl.BlockSpec((H, H), lambda b: (0, 0)),
            pl.BlockSpec((1, H), lambda b: (0, 0)),
            pl.BlockSpec((H, 1), lambda b: (0, 0)),
            pl.BlockSpec((1, 1), lambda b: (0, 0)),
        ],
        out_specs=pl.BlockSpec((1, T, E), lambda b: (b, 0, 0)),
        out_shape=jax.ShapeDtypeStruct((B, T, E), jnp.float32),
    )(batch_embeds, w1, b1.reshape(1, H), w2, b2.reshape(1, H), w3,
      b3.reshape(1, 1))


def _sc_gather2(table_a, idx_a, table_b, idx_b):
    na, d = idx_a.shape[0], table_a.shape[1]
    nb = idx_b.shape[0]
    pa, pb = na // _NW, nb // _NW
    mesh = plsc.VectorSubcoreMesh(core_axis_name="c", subcore_axis_name="s")

    @functools.partial(
        pl.kernel,
        mesh=mesh,
        out_type=[
            jax.ShapeDtypeStruct((na, d), table_a.dtype),
            jax.ShapeDtypeStruct((nb, d), table_b.dtype),
        ],
        scratch_types=[
            pltpu.VMEM((pa,), jnp.int32),
            pltpu.VMEM((pb,), jnp.int32),
            pltpu.VMEM((pa // 2, d), table_a.dtype),
            pltpu.VMEM((pb // 2, d), table_b.dtype),
            pltpu.SemaphoreType.DMA,
            pltpu.SemaphoreType.DMA,
        ],
    )
    def k(ta_hbm, ia_hbm, tb_hbm, ib_hbm, oa_hbm, ob_hbm, ia_v, ib_v, ra_v,
          rb_v, sa, sb):
        wid = lax.axis_index("s") * _NC + lax.axis_index("c")
        pltpu.sync_copy(ia_hbm.at[pl.ds(wid * pa, pa)], ia_v)
        pltpu.sync_copy(ib_hbm.at[pl.ds(wid * pb, pb)], ib_v)
        ha, hb = pa // 2, pb // 2
        for half in range(2):
            cpa = pltpu.async_copy(ta_hbm.at[ia_v.at[pl.ds(half * ha, ha)]],
                                   ra_v, sa)
            cpb = pltpu.async_copy(tb_hbm.at[ib_v.at[pl.ds(half * hb, hb)]],
                                   rb_v, sb)
            cpa.wait()
            pltpu.sync_copy(ra_v, oa_hbm.at[pl.ds(wid * pa + half * ha, ha)])
            cpb.wait()
            pltpu.sync_copy(rb_v, ob_hbm.at[pl.ds(wid * pb + half * hb, hb)])

    return k(table_a, idx_a, table_b, idx_b)


_ROWS = 512


def _mention_body(gs_ref, ge_ref, qs_ref, qe_ref, w1_ref, b1_ref, w2_ref,
                  b2_ref, w3_ref, b3_ref, se_ref, ms_ref):
    gs = gs_ref[...]
    ge = ge_ref[...]
    w = qe_ref[...] - qs_ref[...]
    se = jnp.concatenate([gs, ge, w], axis=1)
    se_ref[...] = se
    h = jnp.maximum(
        jnp.dot(se, w1_ref[...], preferred_element_type=jnp.float32)
        + b1_ref[...], 0.0)
    h = jnp.maximum(
        jnp.dot(h, w2_ref[...], preferred_element_type=jnp.float32)
        + b2_ref[...], 0.0)
    ms_ref[...] = (jnp.dot(h, w3_ref[...], preferred_element_type=jnp.float32)
                   + b3_ref[...])


def _mention(gs, ge, qs, qe, w1, b1, w2, b2, w3, b3):
    n = gs.shape[0]
    row_spec = pl.BlockSpec((_ROWS, E), lambda i: (i, 0))
    return pl.pallas_call(
        _mention_body,
        grid=(n // _ROWS,),
        in_specs=[
            row_spec, row_spec, row_spec, row_spec,
            pl.BlockSpec((3 * E, H), lambda i: (0, 0)),
            pl.BlockSpec((1, H), lambda i: (0, 0)),
            pl.BlockSpec((H, H), lambda i: (0, 0)),
            pl.BlockSpec((1, H), lambda i: (0, 0)),
            pl.BlockSpec((H, 1), lambda i: (0, 0)),
            pl.BlockSpec((1, 1), lambda i: (0, 0)),
        ],
        out_specs=[
            pl.BlockSpec((_ROWS, 3 * E), lambda i: (i, 0)),
            pl.BlockSpec((_ROWS, 1), lambda i: (i, 0)),
        ],
        out_shape=[
            jax.ShapeDtypeStruct((n, 3 * E), jnp.float32),
            jax.ShapeDtypeStruct((n, 1), jnp.float32),
        ],
    )(gs, ge, qs, qe, w1, b1.reshape(1, H), w2, b2.reshape(1, H), w3,
      b3.reshape(1, 1))


def kernel(batch_embeds, span_starts, span_widths, Wa1, ba1, Wa2, ba2, Wa3,
           ba3, Ws1, bs1, Ws2, bs2, Ws3, bs3):
    starts = span_starts.astype(jnp.int32)
    ends = starts + span_widths.astype(jnp.int32)
    boff = (jnp.arange(B, dtype=jnp.int32) * T)[:, None]
    fs = (starts + boff).reshape(-1)
    fe = (ends + boff).reshape(-1)
    idx_emb = jnp.concatenate([fs, fe])
    idx_q = jnp.concatenate([fs, fe + 1])

    emb_flat = batch_embeds.reshape(B * T, E)
    q = _attn_prefix(batch_embeds, Wa1, ba1, Wa2, ba2, Wa3, ba3)
    g_emb, g_q = _sc_gather2(emb_flat, idx_emb, q.reshape(B * T, E), idx_q)

    n = B * S
    span_embeds, scores = _mention(g_emb[:n], g_emb[n:], g_q[:n], g_q[n:],
                                   Ws1, bs1, Ws2, bs2, Ws3, bs3)
    return span_embeds.reshape(B, S, 3 * E), scores.reshape(B, S, 1)


# speedup vs baseline: 16.8074x; 1.1332x over previous
import functools

import jax
import jax.numpy as jnp
from jax import lax
from jax.experimental import pallas as pl
from jax.experimental.pallas import tpu as pltpu
from jax.experimental.pallas import tpu_sc as plsc

B, T, E = 8, 2048, 256
S, WMAX = 512, 10
H = 150
CHUNK = 256

_NC, _NS = 2, 16
_NW = _NC * _NS


def _attn_prefix_body(x_ref, w1_ref, b1_ref, w2_ref, b2_ref, w3_ref, b3_ref,
                      q_ref):
    x = x_ref[0]
    h = jnp.maximum(
        jnp.dot(x, w1_ref[...], preferred_element_type=jnp.float32)
        + b1_ref[...], 0.0)
    h = jnp.maximum(
        jnp.dot(h, w2_ref[...], preferred_element_type=jnp.float32)
        + b2_ref[...], 0.0)
    a = (jnp.dot(h, w3_ref[...], preferred_element_type=jnp.float32)
         + b3_ref[...])
    z = x * a

    tri = (lax.broadcasted_iota(jnp.int32, (CHUNK, CHUNK), 0)
           > lax.broadcasted_iota(jnp.int32, (CHUNK, CHUNK), 1)
           ).astype(jnp.float32)
    carry = jnp.zeros((1, E), jnp.float32)
    for i in range(T // CHUNK):
        zc = z[i * CHUNK:(i + 1) * CHUNK, :]
        q_ref[0, i * CHUNK:(i + 1) * CHUNK, :] = (
            jnp.dot(tri, zc, preferred_element_type=jnp.float32) + carry)
        carry = carry + jnp.sum(zc, axis=0, keepdims=True)


def _attn_prefix(batch_embeds, w1, b1, w2, b2, w3, b3):
    return pl.pallas_call(
        _attn_prefix_body,
        grid=(B,),
        in_specs=[
            pl.BlockSpec((1, T, E), lambda b: (b, 0, 0)),
            pl.BlockSpec((E, H), lambda b: (0, 0)),
            pl.BlockSpec((1, H), lambda b: (0, 0)),
            pl.BlockSpec((H, H), lambda b: (0, 0)),
            pl.BlockSpec((1, H), lambda b: (0, 0)),
            pl.BlockSpec((H, 1), lambda b: (0, 0)),
            pl.BlockSpec((1, 1), lambda b: (0, 0)),
        ],
        out_specs=pl.BlockSpec((1, T, E), lambda b: (b, 0, 0)),
        out_shape=jax.ShapeDtypeStruct((B, T, E), jnp.float32),
    )(batch_embeds, w1, b1.reshape(1, H), w2, b2.reshape(1, H), w3,
      b3.reshape(1, 1))


_PW = (B * S) // _NW


def _sc_gather4(emb_flat, q_flat, starts, widths):
    d = emb_flat.shape[1]
    n = B * S
    mesh = plsc.VectorSubcoreMesh(core_axis_name="c", subcore_axis_name="s")
    row = jax.ShapeDtypeStruct((n, d), emb_flat.dtype)

    @functools.partial(
        pl.kernel,
        mesh=mesh,
        out_type=[row, row, row, row],
        scratch_types=[
            pltpu.VMEM((_PW,), jnp.int32),
            pltpu.VMEM((_PW,), jnp.int32),
            pltpu.VMEM((_PW,), jnp.int32),
            pltpu.VMEM((_PW,), jnp.int32),
            pltpu.VMEM((_PW,), jnp.int32),
            pltpu.VMEM((_PW, d), jnp.float32),
            pltpu.VMEM((_PW, d), jnp.float32),
            pltpu.SemaphoreType.DMA,
            pltpu.SemaphoreType.DMA,
        ],
    )
    def k(emb_hbm, q_hbm, st_hbm, wd_hbm, gs_hbm, ge_hbm, qs_hbm, qe_hbm,
          st_v, wd_v, fs_v, fe_v, fq_v, ra_v, rb_v, sa, sb):
        wid = lax.axis_index("s") * _NC + lax.axis_index("c")
        base = wid * _PW
        boff = (wid // (S // _PW)) * T
        pltpu.sync_copy(st_hbm.at[pl.ds(base, _PW)], st_v)
        pltpu.sync_copy(wd_hbm.at[pl.ds(base, _PW)], wd_v)

        @pl.loop(0, _PW, step=16)
        def _(i):
            s16 = st_v[pl.ds(i, 16)] + boff
            e16 = s16 + wd_v[pl.ds(i, 16)]
            fs_v[pl.ds(i, 16)] = s16
            fe_v[pl.ds(i, 16)] = e16
            fq_v[pl.ds(i, 16)] = e16 + 1

        cp = pltpu.async_copy(emb_hbm.at[fs_v], ra_v, sa)
        cq = pltpu.async_copy(emb_hbm.at[fe_v], rb_v, sb)
        cp.wait()
        pltpu.sync_copy(ra_v, gs_hbm.at[pl.ds(base, _PW)])
        cq.wait()
        pltpu.sync_copy(rb_v, ge_hbm.at[pl.ds(base, _PW)])
        cp = pltpu.async_copy(q_hbm.at[fs_v], ra_v, sa)
        cq = pltpu.async_copy(q_hbm.at[fq_v], rb_v, sb)
        cp.wait()
        pltpu.sync_copy(ra_v, qs_hbm.at[pl.ds(base, _PW)])
        cq.wait()
        pltpu.sync_copy(rb_v, qe_hbm.at[pl.ds(base, _PW)])

    return k(emb_flat, q_flat, starts, widths)


_ROWS = 512


def _mention_body(gs_ref, ge_ref, qs_ref, qe_ref, w1_ref, b1_ref, w2_ref,
                  b2_ref, w3_ref, b3_ref, se_ref, ms_ref):
    gs = gs_ref[...]
    ge = ge_ref[...]
    w = qe_ref[...] - qs_ref[...]
    se = jnp.concatenate([gs, ge, w], axis=1)
    se_ref[...] = se
    h = jnp.maximum(
        jnp.dot(se, w1_ref[...], preferred_element_type=jnp.float32)
        + b1_ref[...], 0.0)
    h = jnp.maximum(
        jnp.dot(h, w2_ref[...], preferred_element_type=jnp.float32)
        + b2_ref[...], 0.0)
    ms_ref[...] = (jnp.dot(h, w3_ref[...], preferred_element_type=jnp.float32)
                   + b3_ref[...])


def _mention(gs, ge, qs, qe, w1, b1, w2, b2, w3, b3):
    n = gs.shape[0]
    row_spec = pl.BlockSpec((_ROWS, E), lambda i: (i, 0))
    return pl.pallas_call(
        _mention_body,
        grid=(n // _ROWS,),
        in_specs=[
            row_spec, row_spec, row_spec, row_spec,
            pl.BlockSpec((3 * E, H), lambda i: (0, 0)),
            pl.BlockSpec((1, H), lambda i: (0, 0)),
            pl.BlockSpec((H, H), lambda i: (0, 0)),
            pl.BlockSpec((1, H), lambda i: (0, 0)),
            pl.BlockSpec((H, 1), lambda i: (0, 0)),
            pl.BlockSpec((1, 1), lambda i: (0, 0)),
        ],
        out_specs=[
            pl.BlockSpec((_ROWS, 3 * E), lambda i: (i, 0)),
            pl.BlockSpec((_ROWS, 1), lambda i: (i, 0)),
        ],
        out_shape=[
            jax.ShapeDtypeStruct((n, 3 * E), jnp.float32),
            jax.ShapeDtypeStruct((n, 1), jnp.float32),
        ],
    )(gs, ge, qs, qe, w1, b1.reshape(1, H), w2, b2.reshape(1, H), w3,
      b3.reshape(1, 1))


def kernel(batch_embeds, span_starts, span_widths, Wa1, ba1, Wa2, ba2, Wa3,
           ba3, Ws1, bs1, Ws2, bs2, Ws3, bs3):
    starts = span_starts.astype(jnp.int32).reshape(-1)
    widths = span_widths.astype(jnp.int32).reshape(-1)

    emb_flat = batch_embeds.reshape(B * T, E)
    q = _attn_prefix(batch_embeds, Wa1, ba1, Wa2, ba2, Wa3, ba3)
    gs, ge, qs, qe = _sc_gather4(emb_flat, q.reshape(B * T, E), starts,
                                 widths)
    span_embeds, scores = _mention(gs, ge, qs, qe,
                                   Ws1, bs1, Ws2, bs2, Ws3, bs3)
    return span_embeds.reshape(B, S, 3 * E), scores.reshape(B, S, 1)
